# baseline (speedup 1.0000x reference)
import jax
import jax.numpy as jnp
import numpy as np
from jax.experimental import pallas as pl
from jax.experimental.pallas import tpu as pltpu

_BLKE = 80000

_DN = (((0,), (0,)), ((), ()))


def _mlp_kernel(x_ref, w1_ref, b1_ref, w2_ref, b2_ref, o_ref):
    i = pl.program_id(0)
    x = x_ref[:, pl.ds(i * _BLKE, _BLKE)]
    b1 = b1_ref[...].reshape(-1, 1)
    b2 = b2_ref[...].reshape(-1, 1)
    h = jax.lax.dot_general(w1_ref[...], x, _DN,
                            preferred_element_type=jnp.float32) + b1
    h = 0.5 * h * (1.0 + jax.lax.erf(h * np.float32(1.0 / np.sqrt(2.0))))
    o_ref[...] = jax.lax.dot_general(w2_ref[...], h, _DN,
                                     preferred_element_type=jnp.float32) + b2


def kernel(node_features, edge_features, edge_index, node_tiers,
           Wq1, Wk1, Wv1, We1, Wo1, Wq2, Wk2, Wv2, We2, Wo2,
           Wc1, bc1, Wc2, bc2):
    E, DE = edge_features.shape
    C = Wc2.shape[1]
    nblk = E // _BLKE

    x_t = edge_features.T

    out_t = pl.pallas_call(
        _mlp_kernel,
        grid=(nblk,),
        in_specs=[
            pl.BlockSpec(memory_space=pltpu.MemorySpace.VMEM),
            pl.BlockSpec((DE, DE), lambda i: (0, 0)),
            pl.BlockSpec((DE,), lambda i: (0,)),
            pl.BlockSpec((DE, C), lambda i: (0, 0)),
            pl.BlockSpec((C,), lambda i: (0,)),
        ],
        out_specs=pl.BlockSpec((C, _BLKE), lambda i: (0, i)),
        out_shape=jax.ShapeDtypeStruct((C, E), jnp.float32),
        compiler_params=pltpu.CompilerParams(
            dimension_semantics=("parallel",),
        ),
    )(x_t, Wc1, bc1, Wc2, bc2)
    return out_t.T


# speedup vs baseline: 1.0780x; 1.0780x over previous
import jax
import jax.numpy as jnp
import numpy as np
from jax.experimental import pallas as pl
from jax.experimental.pallas import tpu as pltpu

_BLKE = 80000

_DN = (((0,), (0,)), ((), ()))


def _mlp_kernel(x_ref, w1_ref, b1_ref, w2_ref, b2_ref, o_ref):
    x = x_ref[...]
    b1 = b1_ref[...].reshape(-1, 1)
    b2 = b2_ref[...].reshape(-1, 1)
    h = jax.lax.dot_general(w1_ref[...], x, _DN,
                            preferred_element_type=jnp.float32) + b1
    h = 0.5 * h * (1.0 + jax.lax.erf(h * np.float32(1.0 / np.sqrt(2.0))))
    o_ref[...] = jax.lax.dot_general(w2_ref[...], h, _DN,
                                     preferred_element_type=jnp.float32) + b2


def kernel(node_features, edge_features, edge_index, node_tiers,
           Wq1, Wk1, Wv1, We1, Wo1, Wq2, Wk2, Wv2, We2, Wo2,
           Wc1, bc1, Wc2, bc2):
    E, DE = edge_features.shape
    C = Wc2.shape[1]
    nblk = E // _BLKE

    x_t = edge_features.T

    out_t = pl.pallas_call(
        _mlp_kernel,
        grid=(nblk,),
        in_specs=[
            pl.BlockSpec((DE, _BLKE), lambda i: (0, i)),
            pl.BlockSpec((DE, DE), lambda i: (0, 0)),
            pl.BlockSpec((DE,), lambda i: (0,)),
            pl.BlockSpec((DE, C), lambda i: (0, 0)),
            pl.BlockSpec((C,), lambda i: (0,)),
        ],
        out_specs=pl.BlockSpec((C, _BLKE), lambda i: (0, i)),
        out_shape=jax.ShapeDtypeStruct((C, E), jnp.float32),
        compiler_params=pltpu.CompilerParams(
            dimension_semantics=("parallel",),
        ),
    )(x_t, Wc1, bc1, Wc2, bc2)
    return out_t.T
